# baseline (speedup 1.0000x reference)
import functools

import jax
import jax.numpy as jnp
from jax import lax
from jax.experimental import pallas as pl
from jax.experimental.pallas import tpu as pltpu
from jax.experimental.pallas import tpu_sc as plsc

_NC = 2
_NS = 16
_NW = _NC * _NS
_LANES = 16
@functools.partial(jax.jit, static_argnames=("rows", "d", "chunk", "slots", "pf"))
def _pos_enc_sc(tok, idx, table, *, rows, d, chunk, slots, pf):
    rpw = rows // _NW
    nchunk = rpw // chunk
    nstep = nchunk // slots
    nvec = d // _LANES
    drain = slots - pf

    mesh = plsc.VectorSubcoreMesh(core_axis_name="c", subcore_axis_name="s")

    @functools.partial(
        pl.kernel,
        mesh=mesh,
        out_type=jax.ShapeDtypeStruct((rows, d), jnp.float32),
        scratch_types=[
            pltpu.VMEM((rpw,), jnp.int32),
            pltpu.VMEM((slots, chunk, d), jnp.float32),
            pltpu.VMEM((slots, chunk, d), jnp.float32),
            [pltpu.SemaphoreType.DMA] * slots,
            [pltpu.SemaphoreType.DMA] * slots,
            [pltpu.SemaphoreType.DMA] * slots,
            pltpu.SemaphoreType.DMA,
        ],
    )
    def k(tok_hbm, idx_hbm, tab_hbm, out_hbm, idx_v, tok_v, row_v,
          tsem, rsem, osem, isem):
        wid = lax.axis_index("s") * _NC + lax.axis_index("c")
        base = wid * rpw

        idx_cp = pltpu.async_copy(idx_hbm.at[pl.ds(base, rpw)], idx_v, isem)

        def start_in(g, slot, rslot):
            r0 = base + g * chunk
            pltpu.async_copy(tok_hbm.at[pl.ds(r0, chunk)], tok_v.at[slot],
                             tsem[slot])
            pltpu.async_copy(tab_hbm.at[idx_v.at[pl.ds(g * chunk, chunk)]],
                             row_v.at[rslot], rsem[rslot])

        def wait_in(g, slot, rslot):
            r0 = base + g * chunk
            pltpu.make_async_copy(tok_hbm.at[pl.ds(r0, chunk)],
                                  tok_v.at[slot], tsem[slot]).wait()
            pltpu.make_async_copy(tab_hbm.at[idx_v.at[pl.ds(g * chunk, chunk)]],
                                  row_v.at[rslot], rsem[rslot]).wait()

        def start_out(g, slot):
            r0 = base + g * chunk
            pltpu.async_copy(tok_v.at[slot], out_hbm.at[pl.ds(r0, chunk)],
                             osem[slot])

        def wait_out(g, slot):
            r0 = base + g * chunk
            pltpu.make_async_copy(tok_v.at[slot],
                                  out_hbm.at[pl.ds(r0, chunk)],
                                  osem[slot]).wait()

        def add_chunk(slot, rslot):
            def add_row(r, c2):
                for j in range(nvec):
                    sl = pl.ds(j * _LANES, _LANES)
                    plsc.addupdate(tok_v.at[slot, r, sl], row_v[rslot, r, sl])
                return c2
            lax.fori_loop(0, chunk, add_row, 0)

        for h in range(pf):
            pltpu.async_copy(tok_hbm.at[pl.ds(base + h * chunk, chunk)],
                             tok_v.at[h], tsem[h])
        idx_cp.wait()
        for h in range(pf):
            pltpu.async_copy(tab_hbm.at[idx_v.at[pl.ds(h * chunk, chunk)]],
                             row_v.at[h], rsem[h])

        def step(t, carry):
            for u in range(slots):
                g = slots * t + u

                @pl.when(g >= drain)
                def _():
                    wait_out(g - drain, (u - drain) % slots)

                @pl.when(g + pf < nchunk)
                def _():
                    start_in(g + pf, (u + pf) % slots, (u + pf) % slots)

                wait_in(g, u, u)
                add_chunk(u, u)
                start_out(g, u)
            return carry

        lax.fori_loop(0, nstep, step, 0)
        for g in range(nstep * slots, nchunk):
            u = g % slots
            wait_out(g - drain, (g - drain) % slots)
            wait_in(g, u, u)
            add_chunk(u, u)
            start_out(g, u)
        for h in range(nchunk - drain, nchunk):
            wait_out(h, h % slots)

    return k(tok, idx, table)


def kernel(tokens, pos_indices, pos_table):
    b, s, d = tokens.shape
    rows = b * s
    tok = tokens.reshape(rows, d)
    idx = pos_indices.reshape(rows).astype(jnp.int32)
    out = _pos_enc_sc(tok, idx, pos_table, rows=rows, d=d,
                      chunk=16, slots=3, pf=2)
    return out.reshape(b, s, d)


# speedup vs baseline: 1.8870x; 1.8870x over previous
import functools

import jax
import jax.numpy as jnp
from jax import lax
from jax.experimental import pallas as pl
from jax.experimental.pallas import tpu as pltpu
from jax.experimental.pallas import tpu_sc as plsc

_NC = 2
_NS = 16
_NW = _NC * _NS
_LANES = 16
@functools.partial(jax.jit, static_argnames=("rows", "d", "chunk", "slots", "pf"))
def _pos_enc_sc(tok, idx, table, *, rows, d, chunk, slots, pf):
    rpw = rows // _NW
    nchunk = rpw // chunk
    nstep = nchunk // slots
    nvec = d // _LANES
    drain = slots - pf

    mesh = plsc.VectorSubcoreMesh(core_axis_name="c", subcore_axis_name="s")

    @functools.partial(
        pl.kernel,
        mesh=mesh,
        out_type=jax.ShapeDtypeStruct((rows, d), jnp.float32),
        scratch_types=[
            pltpu.VMEM((rpw,), jnp.int32),
            pltpu.VMEM((slots, chunk, d), jnp.float32),
            pltpu.VMEM((slots, chunk, d), jnp.float32),
            [pltpu.SemaphoreType.DMA] * slots,
            [pltpu.SemaphoreType.DMA] * slots,
            [pltpu.SemaphoreType.DMA] * slots,
            pltpu.SemaphoreType.DMA,
        ],
    )
    def k(tok_hbm, idx_hbm, tab_hbm, out_hbm, idx_v, tok_v, row_v,
          tsem, rsem, osem, isem):
        wid = lax.axis_index("s") * _NC + lax.axis_index("c")
        base = wid * rpw

        idx_cp = pltpu.async_copy(idx_hbm.at[pl.ds(base, rpw)], idx_v, isem)

        def start_in(g, slot, rslot):
            r0 = base + g * chunk
            pltpu.async_copy(tab_hbm.at[idx_v.at[pl.ds(g * chunk, chunk)]],
                             row_v.at[rslot], rsem[rslot])
            pltpu.async_copy(tok_hbm.at[pl.ds(r0, chunk)], tok_v.at[slot],
                             tsem[slot])

        def wait_in(g, slot, rslot):
            r0 = base + g * chunk
            pltpu.make_async_copy(tok_hbm.at[pl.ds(r0, chunk)],
                                  tok_v.at[slot], tsem[slot]).wait()
            pltpu.make_async_copy(tab_hbm.at[idx_v.at[pl.ds(g * chunk, chunk)]],
                                  row_v.at[rslot], rsem[rslot]).wait()

        def start_out(g, slot):
            r0 = base + g * chunk
            pltpu.async_copy(tok_v.at[slot], out_hbm.at[pl.ds(r0, chunk)],
                             osem[slot])

        def wait_out(g, slot):
            r0 = base + g * chunk
            pltpu.make_async_copy(tok_v.at[slot],
                                  out_hbm.at[pl.ds(r0, chunk)],
                                  osem[slot]).wait()

        def add_chunk(slot, rslot):
            def add_row(r, c2):
                for j in range(nvec):
                    sl = pl.ds(j * _LANES, _LANES)
                    plsc.addupdate(tok_v.at[slot, r, sl], row_v[rslot, r, sl])
                return c2
            lax.fori_loop(0, chunk, add_row, 0)

        for h in range(pf):
            pltpu.async_copy(tok_hbm.at[pl.ds(base + h * chunk, chunk)],
                             tok_v.at[h], tsem[h])
        idx_cp.wait()
        for h in range(pf):
            pltpu.async_copy(tab_hbm.at[idx_v.at[pl.ds(h * chunk, chunk)]],
                             row_v.at[h], rsem[h])

        def step(t, carry):
            for u in range(slots):
                g = slots * t + u

                @pl.when(g >= drain)
                def _():
                    wait_out(g - drain, (u - drain) % slots)

                @pl.when(g + pf < nchunk)
                def _():
                    start_in(g + pf, (u + pf) % slots, (u + pf) % slots)

                wait_in(g, u, u)
                add_chunk(u, u)
                start_out(g, u)
            return carry

        lax.fori_loop(0, nstep, step, 0)
        for g in range(nstep * slots, nchunk):
            u = g % slots
            wait_out(g - drain, (g - drain) % slots)
            wait_in(g, u, u)
            add_chunk(u, u)
            start_out(g, u)
        for h in range(nchunk - drain, nchunk):
            wait_out(h, h % slots)

    return k(tok, idx, table)


def kernel(tokens, pos_indices, pos_table):
    b, s, d = tokens.shape
    rows = b * s
    tok = tokens.reshape(rows, d)
    idx = pos_indices.reshape(rows).astype(jnp.int32)
    out = _pos_enc_sc(tok, idx, pos_table, rows=rows, d=d,
                      chunk=8, slots=4, pf=2)
    return out.reshape(b, s, d)
